# baseline (speedup 1.0000x reference)
import jax
import jax.numpy as jnp
from jax.experimental import pallas as pl

_B = 16
_NPG = 1024
_D = 128
_K = 16
_KL = 127


def _mlp_body(x_ref, w1_ref, b1_ref, w2_ref, b2_ref, o_ref):
    h = jnp.maximum(
        jnp.dot(x_ref[...], w1_ref[...]) + b1_ref[...], 0.0)
    o_ref[...] = jnp.dot(h, w2_ref[...]) + b2_ref[...]


def _knn_batched(feat, k):
    f = feat.reshape(_B, _NPG, -1)
    sq = jnp.sum(f * f, axis=-1)
    d2 = sq[:, :, None] + sq[:, None, :] - 2.0 * jnp.einsum('bnd,bmd->bnm', f, f)
    d2 = d2 + jnp.eye(_NPG, dtype=f.dtype)[None, :, :] * 1e10
    _, idx = jax.lax.top_k(-d2, k)
    offs = (jnp.arange(_B) * _NPG)[:, None, None]
    src = (idx + offs).reshape(-1)
    dst = jnp.repeat(jnp.arange(_B * _NPG), k)
    return jnp.stack([src, dst])


def kernel(x, pos, batch, W1, b1, W2, b2, t):
    edge_index = _knn_batched(pos, _K)
    edges_large = _knn_batched(x, _KL)

    emb = pl.pallas_call(
        _mlp_body,
        out_shape=jax.ShapeDtypeStruct((_B * _NPG, 20), jnp.float32),
        grid=(8,),
        in_specs=[
            pl.BlockSpec((2048, _D), lambda i: (i, 0)),
            pl.BlockSpec((_D, _D), lambda i: (0, 0)),
            pl.BlockSpec((1, _D), lambda i: (0, 0)),
            pl.BlockSpec((_D, 20), lambda i: (0, 0)),
            pl.BlockSpec((1, 20), lambda i: (0, 0)),
        ],
        out_specs=pl.BlockSpec((2048, 20), lambda i: (i, 0)),
    )(x, W1, b1.reshape(1, _D), W2, b2.reshape(1, 20))

    kr = jax.random.key(1)
    rand_scores = jax.random.uniform(jax.random.fold_in(kr, 0), emb.shape, dtype=emb.dtype) * 1e-4
    emb = emb + rand_scores
    diff = emb[edges_large[0]] - emb[edges_large[1]]
    dist = jnp.linalg.norm(diff, axis=1)
    p = jnp.exp(-t[0] * dist ** 2).reshape(-1, _KL)
    u = jax.random.uniform(jax.random.fold_in(kr, 1), p.shape, dtype=p.dtype)
    gumbel = -jnp.log(-jnp.log(u + 1e-20) + 1e-20)
    noisy_logits = jnp.log(p + 1e-20) + gumbel
    top_v, top_i = jax.lax.top_k(noisy_logits, _K)
    top_v = jax.nn.softmax(top_v, axis=1)
    top_i = top_i + jnp.arange(top_i.shape[0])[:, None] * _KL
    top_i = top_i.reshape(-1)
    top_v = top_v.reshape(-1)
    edges_sparse = edges_large[:, top_i]
    edges_sparse_v = jnp.stack([top_v, edges_sparse[1, :].astype(top_v.dtype)], axis=0)
    edge_index_out = jnp.concatenate([edges_sparse, edge_index], axis=1)
    return (edge_index_out, edges_sparse, edges_sparse_v)


# speedup vs baseline: 3.1671x; 3.1671x over previous
import jax
import jax.numpy as jnp
from jax.experimental import pallas as pl

_B = 16
_NPG = 1024
_D = 128
_K = 16
_KL = 127
_R = 256
_NBLK = _NPG // _R


def _mlp_body(x_ref, w1_ref, b1_ref, w2_ref, b2_ref, o_ref):
    h = jnp.maximum(jnp.dot(x_ref[...], w1_ref[...]) + b1_ref[...], 0.0)
    o_ref[...] = jnp.dot(h, w2_ref[...]) + b2_ref[...]


def _extract_topk(v, payload, n_iter, width):
    rows = v.shape[0]
    lane = jax.lax.broadcasted_iota(jnp.int32, (rows, width), 1)
    slot = jax.lax.broadcasted_iota(jnp.int32, (rows, 128), 1)
    big = jnp.int32(1 << 30)
    inf = jnp.float32(jnp.inf)

    def body(s, carry):
        v, idx_acc, pay_acc = carry
        m = jnp.min(v, axis=1, keepdims=True)
        jcand = jnp.where(v == m, lane, big)
        j = jnp.min(jcand, axis=1, keepdims=True)
        sel = lane == j
        if payload is not None:
            pay = jnp.sum(jnp.where(sel, payload, 0.0), axis=1, keepdims=True)
            pay_acc = jnp.where(slot == s, pay, pay_acc)
        v = jnp.where(sel, inf, v)
        idx_acc = jnp.where(slot == s, j, idx_acc)
        return v, idx_acc, pay_acc

    idx0 = jnp.zeros((rows, 128), jnp.int32)
    pay0 = jnp.zeros((rows, 128), jnp.float32)
    _, idx_acc, pay_acc = jax.lax.fori_loop(0, n_iter, body, (v, idx0, pay0))
    return idx_acc, pay_acc


def _graph_body(x_ref, xgT_ref, sqx_r_ref, sqx_c_ref,
                emb_ref, embT_ref, sqe_r_ref, sqe_c_ref,
                pos_ref, posT_ref, sqp_r_ref, sqp_c_ref,
                gum_ref, t_ref,
                srcx_ref, val_ref, srcp_ref):
    i = pl.program_id(0)
    rb = i % _NBLK
    t = t_ref[0, 0]
    dn = (((1,), (0,)), ((), ()))
    lane1024 = jax.lax.broadcasted_iota(jnp.int32, (_R, _NPG), 1)
    row_g = jax.lax.broadcasted_iota(jnp.int32, (_R, 1), 0) + rb * _R
    diag = lane1024 == row_g

    dotx = jax.lax.dot_general(x_ref[...], xgT_ref[0], dn,
                               preferred_element_type=jnp.float32)
    d2x = (sqx_r_ref[0] + sqx_c_ref[0]) - 2.0 * dotx
    d2x = jnp.where(diag, d2x + 1e10, d2x)

    dote = jax.lax.dot_general(emb_ref[...], embT_ref[0], dn,
                               preferred_element_type=jnp.float32)
    de2 = (sqe_r_ref[0] + sqe_c_ref[0]) - 2.0 * dote

    idx127, de2_127 = _extract_topk(d2x, de2, _KL, _NPG)

    p = jnp.exp(-t * de2_127)
    noisy = jnp.log(p + 1e-20) + gum_ref[...]
    slot = jax.lax.broadcasted_iota(jnp.int32, (_R, 128), 1)
    noisy = jnp.where(slot < _KL, noisy, -jnp.inf)

    lane128 = slot
    big = jnp.int32(1 << 30)
    ninf = jnp.float32(-jnp.inf)

    def body2(s, carry):
        nv, v_acc, src_acc = carry
        m = jnp.max(nv, axis=1, keepdims=True)
        jcand = jnp.where(nv == m, lane128, big)
        j = jnp.min(jcand, axis=1, keepdims=True)
        sel = lane128 == j
        val = jnp.sum(jnp.where(sel, noisy, 0.0), axis=1, keepdims=True)
        src = jnp.sum(jnp.where(sel, idx127, 0), axis=1, keepdims=True)
        v_acc = jnp.where(slot == s, val, v_acc)
        src_acc = jnp.where(slot == s, src, src_acc)
        nv = jnp.where(sel, ninf, nv)
        return nv, v_acc, src_acc

    v0 = jnp.zeros((_R, 128), jnp.float32)
    s0 = jnp.zeros((_R, 128), jnp.int32)
    _, v_acc, src_acc = jax.lax.fori_loop(0, _K, body2, (noisy, v0, s0))

    topv = v_acc[:, :_K]
    mx = jnp.max(topv, axis=1, keepdims=True)
    e = jnp.exp(topv - mx)
    topv = e / jnp.sum(e, axis=1, keepdims=True)

    goff = (i // _NBLK) * _NPG
    srcx_ref[...] = src_acc[:, :_K] + goff
    val_ref[...] = topv

    dotp = jax.lax.dot_general(pos_ref[...], posT_ref[0], dn,
                               preferred_element_type=jnp.float32)
    d2p = (sqp_r_ref[0] + sqp_c_ref[0]) - 2.0 * dotp
    d2p = jnp.where(diag, d2p + 1e10, d2p)
    idxp, _ = _extract_topk(d2p, None, _K, _NPG)
    srcp_ref[...] = idxp[:, :_K] + goff


def _make_call(interpret=False):
    nb = _B * _NBLK
    bg = lambda i: (i // _NBLK, 0, 0)
    br = lambda i: (i, 0)
    br3 = lambda i: (i, 0, 0)
    return pl.pallas_call(
        _graph_body,
        out_shape=(
            jax.ShapeDtypeStruct((_B * _NPG, _K), jnp.int32),
            jax.ShapeDtypeStruct((_B * _NPG, _K), jnp.float32),
            jax.ShapeDtypeStruct((_B * _NPG, _K), jnp.int32),
        ),
        grid=(nb,),
        in_specs=[
            pl.BlockSpec((_R, _D), br),
            pl.BlockSpec((1, _D, _NPG), bg),
            pl.BlockSpec((1, _R, 1), br3),
            pl.BlockSpec((1, 1, _NPG), bg),
            pl.BlockSpec((_R, 20), br),
            pl.BlockSpec((1, 20, _NPG), bg),
            pl.BlockSpec((1, _R, 1), br3),
            pl.BlockSpec((1, 1, _NPG), bg),
            pl.BlockSpec((_R, 3), br),
            pl.BlockSpec((1, 3, _NPG), bg),
            pl.BlockSpec((1, _R, 1), br3),
            pl.BlockSpec((1, 1, _NPG), bg),
            pl.BlockSpec((_R, 128), br),
            pl.BlockSpec((1, 1), lambda i: (0, 0)),
        ],
        out_specs=(
            pl.BlockSpec((_R, _K), br),
            pl.BlockSpec((_R, _K), br),
            pl.BlockSpec((_R, _K), br),
        ),
        interpret=interpret,
    )


def kernel(x, pos, batch, W1, b1, W2, b2, t, interpret=False):
    n = _B * _NPG
    emb = pl.pallas_call(
        _mlp_body,
        out_shape=jax.ShapeDtypeStruct((n, 20), jnp.float32),
        grid=(8,),
        in_specs=[
            pl.BlockSpec((2048, _D), lambda i: (i, 0)),
            pl.BlockSpec((_D, _D), lambda i: (0, 0)),
            pl.BlockSpec((1, _D), lambda i: (0, 0)),
            pl.BlockSpec((_D, 20), lambda i: (0, 0)),
            pl.BlockSpec((1, 20), lambda i: (0, 0)),
        ],
        out_specs=pl.BlockSpec((2048, 20), lambda i: (i, 0)),
        interpret=interpret,
    )(x, W1, b1.reshape(1, _D), W2, b2.reshape(1, 20))

    kr = jax.random.key(1)
    rand_scores = jax.random.uniform(jax.random.fold_in(kr, 0), emb.shape,
                                     dtype=emb.dtype) * 1e-4
    emb = emb + rand_scores
    u = jax.random.uniform(jax.random.fold_in(kr, 1), (n, _KL), dtype=jnp.float32)
    gum = -jnp.log(-jnp.log(u + 1e-20) + 1e-20)
    gum = jnp.concatenate([gum, jnp.zeros((n, 1), jnp.float32)], axis=1)

    xg = x.reshape(_B, _NPG, _D)
    eg = emb.reshape(_B, _NPG, 20)
    pg = pos.reshape(_B, _NPG, 3)
    sqx = jnp.sum(xg * xg, axis=-1)
    sqe = jnp.sum(eg * eg, axis=-1)
    sqp = jnp.sum(pg * pg, axis=-1)

    srcx, topv, srcp = _make_call(interpret)(
        x, xg.transpose(0, 2, 1), sqx.reshape(_B * _NBLK, _R, 1),
        sqx.reshape(_B, 1, _NPG),
        emb, eg.transpose(0, 2, 1), sqe.reshape(_B * _NBLK, _R, 1),
        sqe.reshape(_B, 1, _NPG),
        pos, pg.transpose(0, 2, 1), sqp.reshape(_B * _NBLK, _R, 1),
        sqp.reshape(_B, 1, _NPG),
        gum, t.reshape(1, 1),
    )

    dst = jnp.repeat(jnp.arange(n, dtype=jnp.int32), _K)
    edges_sparse = jnp.stack([srcx.reshape(-1), dst])
    edge_index = jnp.stack([srcp.reshape(-1), dst])
    topv_f = topv.reshape(-1)
    edges_sparse_v = jnp.stack([topv_f, dst.astype(jnp.float32)], axis=0)
    edge_index_out = jnp.concatenate([edges_sparse, edge_index], axis=1)
    return (edge_index_out, edges_sparse, edges_sparse_v)
